# Initial kernel scaffold; baseline (speedup 1.0000x reference)
import jax
import jax.numpy as jnp
from jax.experimental import pallas as pl


def kernel(braids, initial_state, thetas, ln_gamma, ln_beta, w1, b1, w2, b2, w3, b3):
    raise NotImplementedError("write your pallas kernel here")


# speedup vs baseline: 13.4089x; 13.4089x over previous
import jax
import jax.numpy as jnp
from jax.experimental import pallas as pl
from jax.experimental.pallas import tpu as pltpu

_B = 1024
_L = 20
_H = 128


def _knot_body(br_ref, init_ref, th_ref, g_ref, bt_ref,
               w1_ref, b1_ref, w2_ref, b2_ref, w3_ref, b3_ref,
               out_ref, m_ref):
    f32 = jnp.float32
    iot = jax.lax.broadcasted_iota(jnp.int32, (32, _B), 0)
    m_ref[...] = jnp.where((iot % 4) == (iot // 8), 1.0, 0.0).astype(f32)
    trig = []
    for ppp in range(3):
        cs = []
        for fn in (jnp.cos, jnp.sin):
            rows = [jnp.broadcast_to(fn(th_ref[l:l + 1, ppp:ppp + 1]), (4, 1))
                    for l in range(2)]
            cs.append(jnp.concatenate(rows, axis=0))
        trig.append(cs)
    for t in range(_L):
        gen = br_ref[t:t + 1, :]
        sgn = jnp.where(gen > 0, 1.0, -1.0).astype(f32)
        absg = jnp.abs(gen)
        for ppp in range(3):
            active = absg == (ppp + 1)
            c8, s8 = trig[ppp]
            c = jnp.where(active, c8, 1.0)
            s = jnp.where(active, sgn * s8, 0.0)
            u = m_ref[pl.ds(8 * ppp, 8), :]
            v = m_ref[pl.ds(8 * ppp + 8, 8), :]
            m_ref[pl.ds(8 * ppp, 8), :] = c * u - s * v
            m_ref[pl.ds(8 * ppp + 8, 8), :] = s * u + c * v
    mm = m_ref[...]
    prev = None
    for layer in range(2):
        news = []
        for i in range(4):
            acc = None
            for j in range(4):
                r = i * 8 + layer * 4 + j
                mrow = mm[r:r + 1, :]
                col = init_ref[:, j:j + 1] if layer == 0 else prev[j]
                term = col * mrow
                acc = term if acc is None else acc + term
            news.append(acc)
        gcol = g_ref[:, layer:layer + 1]
        bcol = bt_ref[:, layer:layer + 1]
        prev = []
        for i in range(4):
            x = news[i]
            mean = jnp.mean(x, axis=0, keepdims=True)
            var = jnp.mean((x - mean) ** 2, axis=0, keepdims=True)
            prev.append((x - mean) / jnp.sqrt(var + 1e-5) * gcol + bcol)
    flat = jnp.concatenate(prev, axis=0)
    h1 = jnp.dot(w1_ref[...], flat, preferred_element_type=f32) + b1_ref[...]
    h1 = jnp.maximum(h1, 0.0)
    h2 = jnp.dot(w2_ref[...], h1, preferred_element_type=f32) + b2_ref[...]
    h2 = jnp.maximum(h2, 0.0)
    out = jnp.dot(w3_ref[...], h2, preferred_element_type=f32) + b3_ref[...]
    out_ref[...] = jnp.concatenate(
        [jax.nn.sigmoid(out[0:1, :]), out[1:, :]], axis=0)


def kernel(braids, initial_state, thetas, ln_gamma, ln_beta,
           w1, b1, w2, b2, w3, b3):
    f32 = jnp.float32
    braidsT = braids.T
    initT = initial_state.T.astype(f32)
    thp = jnp.zeros((8, 128), f32).at[:2, :3].set(thetas)
    gT = ln_gamma.T
    btT = ln_beta.T
    b1c = b1.reshape(128, 1)
    b2c = b2.reshape(64, 1)
    w3p = jnp.zeros((8, 64), f32).at[:2, :].set(w3)
    b3p = jnp.zeros((8, 1), f32).at[:2, 0].set(b3)
    out = pl.pallas_call(
        _knot_body,
        out_shape=jax.ShapeDtypeStruct((8, _B), f32),
        scratch_shapes=[pltpu.VMEM((32, _B), f32)],
    )(braidsT, initT, thp, gT, btT, w1, b1c, w2, b2c, w3p, b3p)
    return out[0, :], out[1, :]


# speedup vs baseline: 22.1220x; 1.6498x over previous
import jax
import jax.numpy as jnp
from jax.experimental import pallas as pl
from jax.experimental.pallas import tpu as pltpu

_B = 1024
_L = 20
_H = 128


def _knot_body(br_ref, init_ref, th_ref, g_ref, bt_ref,
               w1_ref, b1_ref, w2_ref, b2_ref, w3_ref, b3_ref,
               out_ref, m_ref):
    f32 = jnp.float32
    braidsT = jnp.transpose(br_ref[...])
    iot = jax.lax.broadcasted_iota(jnp.int32, (32, _B), 0)
    m_ref[...] = jnp.where((iot % 4) == (iot // 8), 1.0, 0.0).astype(f32)
    trig = []
    for ppp in range(3):
        cs = []
        for fn in (jnp.cos, jnp.sin):
            rows = [jnp.broadcast_to(fn(th_ref[l:l + 1, ppp:ppp + 1]), (4, 1))
                    for l in range(2)]
            cs.append(jnp.concatenate(rows, axis=0))
        trig.append(cs)
    for t in range(_L):
        gen = braidsT[t:t + 1, :]
        sgn = jnp.where(gen > 0, 1.0, -1.0).astype(f32)
        absg = jnp.abs(gen)
        for ppp in range(3):
            active = absg == (ppp + 1)
            c8, s8 = trig[ppp]
            c = jnp.where(active, c8, 1.0)
            s = jnp.where(active, sgn * s8, 0.0)
            u = m_ref[pl.ds(8 * ppp, 8), :]
            v = m_ref[pl.ds(8 * ppp + 8, 8), :]
            m_ref[pl.ds(8 * ppp, 8), :] = c * u - s * v
            m_ref[pl.ds(8 * ppp + 8, 8), :] = s * u + c * v
    mm = m_ref[...]
    initT = jnp.transpose(init_ref[...])
    gT = jnp.transpose(g_ref[...])
    btT = jnp.transpose(bt_ref[...])
    prev = None
    for layer in range(2):
        news = []
        for i in range(4):
            acc = None
            for j in range(4):
                r = i * 8 + layer * 4 + j
                mrow = mm[r:r + 1, :]
                col = initT[:, j:j + 1] if layer == 0 else prev[j]
                term = col * mrow
                acc = term if acc is None else acc + term
            news.append(acc)
        gcol = gT[:, layer:layer + 1]
        bcol = btT[:, layer:layer + 1]
        prev = []
        for i in range(4):
            x = news[i]
            mean = jnp.mean(x, axis=0, keepdims=True)
            var = jnp.mean((x - mean) ** 2, axis=0, keepdims=True)
            prev.append((x - mean) / jnp.sqrt(var + 1e-5) * gcol + bcol)
    b1c = jnp.transpose(jnp.reshape(b1_ref[...], (1, 128)))
    b2c = jnp.transpose(jnp.reshape(b2_ref[...], (1, 64)))
    b3c = jnp.transpose(jnp.reshape(b3_ref[...], (1, 2)))
    flat = jnp.concatenate(prev, axis=0)
    h1 = jnp.dot(w1_ref[...], flat, preferred_element_type=f32) + b1c
    h1 = jnp.maximum(h1, 0.0)
    h2 = jnp.dot(w2_ref[...], h1, preferred_element_type=f32) + b2c
    h2 = jnp.maximum(h2, 0.0)
    out = jnp.dot(w3_ref[...], h2, preferred_element_type=f32) + b3c
    out_ref[...] = jnp.concatenate(
        [jax.nn.sigmoid(out[0:1, :]), out[1:2, :]], axis=0)


def kernel(braids, initial_state, thetas, ln_gamma, ln_beta,
           w1, b1, w2, b2, w3, b3):
    out = pl.pallas_call(
        _knot_body,
        out_shape=jax.ShapeDtypeStruct((2, _B), jnp.float32),
        scratch_shapes=[pltpu.VMEM((32, _B), jnp.float32)],
    )(braids, initial_state, thetas, ln_gamma, ln_beta,
      w1, b1, w2, b2, w3, b3)
    return out[0, :], out[1, :]


# speedup vs baseline: 25.8071x; 1.1666x over previous
import jax
import jax.numpy as jnp
from jax.experimental import pallas as pl
from jax.experimental.pallas import tpu as pltpu

_B = 1024
_L = 20
_H = 128


def _knot_body(br_ref, init_ref, th_ref, g_ref, bt_ref,
               w1_ref, b1_ref, w2_ref, b2_ref, w3_ref, b3_ref,
               o1_ref, o2_ref, m_ref):
    f32 = jnp.float32
    braidsT = jnp.transpose(br_ref[...])
    iot = jax.lax.broadcasted_iota(jnp.int32, (32, _B), 0)
    m_ref[...] = jnp.where((iot % 4) == (iot // 8), 1.0, 0.0).astype(f32)
    trig = []
    for ppp in range(3):
        cs = []
        for fn in (jnp.cos, jnp.sin):
            rows = [jnp.broadcast_to(fn(th_ref[l:l + 1, ppp:ppp + 1]), (4, 1))
                    for l in range(2)]
            cs.append(jnp.concatenate(rows, axis=0))
        trig.append(cs)
    for t in range(_L):
        gen = braidsT[t:t + 1, :]
        sgn = jnp.where(gen > 0, 1.0, -1.0).astype(f32)
        absg = jnp.abs(gen)
        for ppp in range(3):
            active = absg == (ppp + 1)
            c8, s8 = trig[ppp]
            c = jnp.where(active, c8, 1.0)
            s = jnp.where(active, sgn * s8, 0.0)
            u = m_ref[pl.ds(8 * ppp, 8), :]
            v = m_ref[pl.ds(8 * ppp + 8, 8), :]
            m_ref[pl.ds(8 * ppp, 8), :] = c * u - s * v
            m_ref[pl.ds(8 * ppp + 8, 8), :] = s * u + c * v
    mm = m_ref[...]
    initT = jnp.transpose(init_ref[...])
    gT = jnp.transpose(g_ref[...])
    btT = jnp.transpose(bt_ref[...])
    prev = None
    for layer in range(2):
        news = []
        for i in range(4):
            acc = None
            for j in range(4):
                r = i * 8 + layer * 4 + j
                mrow = mm[r:r + 1, :]
                col = initT[:, j:j + 1] if layer == 0 else prev[j]
                term = col * mrow
                acc = term if acc is None else acc + term
            news.append(acc)
        gcol = gT[:, layer:layer + 1]
        bcol = btT[:, layer:layer + 1]
        prev = []
        for i in range(4):
            x = news[i]
            mean = jnp.mean(x, axis=0, keepdims=True)
            var = jnp.mean((x - mean) ** 2, axis=0, keepdims=True)
            prev.append((x - mean) / jnp.sqrt(var + 1e-5) * gcol + bcol)
    b1c = jnp.transpose(jnp.reshape(b1_ref[...], (1, 128)))
    b2c = jnp.transpose(jnp.reshape(b2_ref[...], (1, 64)))
    b3c = jnp.transpose(jnp.reshape(b3_ref[...], (1, 2)))
    flat = jnp.concatenate(prev, axis=0)
    h1 = jnp.dot(w1_ref[...], flat, preferred_element_type=f32) + b1c
    h1 = jnp.maximum(h1, 0.0)
    h2 = jnp.dot(w2_ref[...], h1, preferred_element_type=f32) + b2c
    h2 = jnp.maximum(h2, 0.0)
    out = jnp.dot(w3_ref[...], h2, preferred_element_type=f32) + b3c
    o1_ref[...] = jnp.reshape(jax.nn.sigmoid(out[0:1, :]), (_B,))
    o2_ref[...] = jnp.reshape(out[1:2, :], (_B,))


def kernel(braids, initial_state, thetas, ln_gamma, ln_beta,
           w1, b1, w2, b2, w3, b3):
    o1, o2 = pl.pallas_call(
        _knot_body,
        out_shape=[jax.ShapeDtypeStruct((_B,), jnp.float32),
                   jax.ShapeDtypeStruct((_B,), jnp.float32)],
        scratch_shapes=[pltpu.VMEM((32, _B), jnp.float32)],
    )(braids, initial_state, thetas, ln_gamma, ln_beta,
      w1, b1, w2, b2, w3, b3)
    return o1, o2
